# Initial kernel scaffold; baseline (speedup 1.0000x reference)
import jax
import jax.numpy as jnp
from jax.experimental import pallas as pl


def kernel(inputs, edge_index, edge_attr, gcn_w1, gcn_b1, gcn_w2, gcn_b2, gcn_w3, gcn_b3, gcn_w4, gcn_b4, tr_in_w, tr_in_b, tr_out_w, tr_out_b, tr_l1_w, tr_l1_b, tr_l2_w, tr_l2_b, tr_ln1_w, tr_ln1_b, tr_ln2_w, tr_ln2_b, out_w, out_b):
    raise NotImplementedError("write your pallas kernel here")


# speedup vs baseline: 7.8730x; 7.8730x over previous
import jax
import jax.numpy as jnp
import numpy as np
from jax import lax
from jax.experimental import pallas as pl
from jax.experimental.pallas import tpu as pltpu

B = 32
SEQ = 96
ENC = 120
E = 1920
LBL = 48
D = 1920
NHEAD = 3
DH = D // NHEAD
NLAYERS = 4
DFF = 512
NP = 128
FP = 128
HC = 960
NST = 11


def _gcn_body(x_ref, src_ref, dst_ref, ew_ref, w1_ref, b1_ref, w2_ref, b2_ref,
              w3_ref, b3_ref, w4_ref, b4_ref, out_ref):
    src = src_ref[...]
    dst = dst_ref[...]
    ew = ew_ref[...]
    lane = lax.broadcasted_iota(jnp.int32, (E, NP), 1).astype(jnp.float32)
    src_oh = (src == lane).astype(jnp.float32)
    dst_oh = (dst == lane).astype(jnp.float32)
    deg = lax.dot_general(ew, dst_oh, (((0,), (0,)), ((), ())),
                          preferred_element_type=jnp.float32)
    deg = deg + 1.0
    dinv = jnp.where(deg > 0, lax.rsqrt(deg), 0.0)
    dinv_src = lax.dot_general(src_oh, dinv, (((1,), (1,)), ((), ())),
                               preferred_element_type=jnp.float32)
    dinv_dst = lax.dot_general(dst_oh, dinv, (((1,), (1,)), ((), ())),
                               preferred_element_type=jnp.float32)
    norm = dinv_src * ew * dinv_dst
    a_mat = lax.dot_general(dst_oh * norm, src_oh, (((0,), (0,)), ((), ())),
                            preferred_element_type=jnp.float32)
    r_i = lax.broadcasted_iota(jnp.int32, (NP, NP), 0)
    c_i = lax.broadcasted_iota(jnp.int32, (NP, NP), 1)
    eye = (r_i == c_i).astype(jnp.float32)
    a_mat = a_mat + eye * (dinv * dinv)

    x3 = x_ref[...]
    x2 = x3.reshape(NP * B, FP)
    for w_ref, b_ref in ((w1_ref, b1_ref), (w2_ref, b2_ref),
                         (w3_ref, b3_ref), (w4_ref, b4_ref)):
        w = w_ref[...]
        bvec = b_ref[...]
        y = lax.dot_general(x2, w, (((1,), (1,)), ((), ())),
                            preferred_element_type=jnp.float32)
        yv = y.reshape(NP, B * FP)
        z = lax.dot_general(a_mat, yv, (((1,), (0,)), ((), ())),
                            preferred_element_type=jnp.float32)
        x2 = jnp.maximum(z.reshape(NP * B, FP) + bvec, 0.0)
    out_ref[...] = x2.reshape(NP, B, FP)


def _graph_gcn_call(x3, src_c, dst_c, ew_c, w1p, b1p, w2p, b2p, w3p, b3p,
                    w4p, b4p):
    return pl.pallas_call(
        _gcn_body,
        out_shape=jax.ShapeDtypeStruct((NP, B, FP), jnp.float32),
    )(x3, src_c, dst_c, ew_c, w1p, b1p, w2p, b2p, w3p, b3p, w4p, b4p)


def _ln(x, w, b):
    mu = jnp.mean(x, axis=-1, keepdims=True)
    var = jnp.mean((x - mu) ** 2, axis=-1, keepdims=True)
    return (x - mu) / jnp.sqrt(var + 1e-5) * w + b


def _tr_body(h_ref, win_ref, wout_ref, w1_ref, w2_ref, inb_ref, outb_ref,
             l1b_ref, l2b_ref, ln1w_ref, ln1b_ref, ln2w_ref, ln2b_ref,
             ow_ref, ob_ref, out_ref, h_s, qkv_s, o_s, pj_s, ff_s):
    g = pl.program_id(0)
    s = g % NST

    @pl.when(g == 0)
    def _init():
        h_s[...] = h_ref[...]

    @pl.when(s < 6)
    def _qkv():
        part = lax.dot_general(h_s[...], win_ref[0],
                               (((1,), (1,)), ((), ())),
                               preferred_element_type=jnp.float32)
        qkv_s[pl.ds(s, 1)] = (part + inb_ref[0])[None]

    @pl.when(s == 6)
    def _attn():
        scale = jnp.float32(np.sqrt(DH).astype(np.float32))
        q = jnp.concatenate([qkv_s[0], qkv_s[1]], axis=1)
        k = jnp.concatenate([qkv_s[2], qkv_s[3]], axis=1)
        v = jnp.concatenate([qkv_s[4], qkv_s[5]], axis=1)
        outs = []
        for hh in range(NHEAD):
            qh = q[:, hh * DH:(hh + 1) * DH]
            kh = k[:, hh * DH:(hh + 1) * DH]
            vh = v[:, hh * DH:(hh + 1) * DH]
            logits = lax.dot_general(qh, kh, (((1,), (1,)), ((), ())),
                                     preferred_element_type=jnp.float32)
            logits = logits / scale
            m = jnp.max(logits, axis=-1, keepdims=True)
            e = jnp.exp(logits - m)
            att = e / jnp.sum(e, axis=-1, keepdims=True)
            outs.append(lax.dot_general(att, vh, (((1,), (0,)), ((), ())),
                                        preferred_element_type=jnp.float32))
        o_s[...] = jnp.concatenate(outs, axis=1)

    @pl.when((s == 7) | (s == 8))
    def _proj():
        pj = lax.dot_general(o_s[...], wout_ref[0], (((1,), (1,)), ((), ())),
                             preferred_element_type=jnp.float32)
        pj_s[pl.ds(s - 7, 1)] = (pj + outb_ref[0])[None]

        @pl.when(s == 8)
        def _res1():
            pj_full = jnp.concatenate([pj_s[0], pj_s[1]], axis=1)
            h_s[...] = _ln(h_s[...] + pj_full, ln1w_ref[0], ln1b_ref[0])

    @pl.when(s == 9)
    def _ff1():
        hid = lax.dot_general(h_s[...], w1_ref[0], (((1,), (1,)), ((), ())),
                              preferred_element_type=jnp.float32)
        ff_s[...] = jnp.maximum(hid + l1b_ref[0], 0.0)

    @pl.when(s == 10)
    def _ff2():
        ff2 = lax.dot_general(ff_s[...], w2_ref[0], (((1,), (1,)), ((), ())),
                              preferred_element_type=jnp.float32)
        ff2 = ff2 + l2b_ref[0]
        h2 = _ln(h_s[...] + ff2, ln2w_ref[0], ln2b_ref[0])
        h_s[...] = h2

        @pl.when(g == NST * NLAYERS - 1)
        def _final():
            res = lax.dot_general(h2, ow_ref[...], (((1,), (1,)), ((), ())),
                                  preferred_element_type=jnp.float32)
            out_ref[...] = res + ob_ref[...]


def _transformer_call(h, win_r, wout_r, w1, w2, inb_r, outb_r, l1b_r, l2b_r,
                      ln1w_r, ln1b_r, ln2w_r, ln2b_r, ow, ob2):
    nsteps = NST * NLAYERS
    const2 = lambda g: (0, 0)
    win_idx = lambda g: (6 * (g // NST) + jnp.minimum(g % NST, 5), 0, 0)
    wout_idx = lambda g: (jnp.where(
        g % NST >= 8, 2 * (g // NST) + 1,
        jnp.where(g % NST >= 7, 2 * (g // NST),
                  jnp.maximum(2 * (g // NST) - 1, 0))), 0, 0)
    w1_idx = lambda g: (jnp.where(g % NST >= 9, g // NST,
                                  jnp.maximum(g // NST - 1, 0)), 0, 0)
    w2_idx = lambda g: (jnp.where(g % NST >= 10, g // NST,
                                  jnp.maximum(g // NST - 1, 0)), 0, 0)
    ln1_idx = lambda g: (jnp.where(g % NST >= 8, g // NST,
                                   jnp.maximum(g // NST - 1, 0)), 0, 0)
    in_specs = [
        pl.BlockSpec((B, D), const2),
        pl.BlockSpec((1, HC, D), win_idx),
        pl.BlockSpec((1, HC, D), wout_idx),
        pl.BlockSpec((1, DFF, D), w1_idx),
        pl.BlockSpec((1, D, DFF), w2_idx),
        pl.BlockSpec((1, 1, HC), win_idx),
        pl.BlockSpec((1, 1, HC), wout_idx),
        pl.BlockSpec((1, 1, DFF), w1_idx),
        pl.BlockSpec((1, 1, D), w2_idx),
        pl.BlockSpec((1, 1, D), ln1_idx),
        pl.BlockSpec((1, 1, D), ln1_idx),
        pl.BlockSpec((1, 1, D), w2_idx),
        pl.BlockSpec((1, 1, D), w2_idx),
        pl.BlockSpec((LBL, D), const2),
        pl.BlockSpec((1, LBL), const2),
    ]
    return pl.pallas_call(
        _tr_body,
        grid=(nsteps,),
        in_specs=in_specs,
        out_specs=pl.BlockSpec((B, LBL), const2),
        out_shape=jax.ShapeDtypeStruct((B, LBL), jnp.float32),
        scratch_shapes=[
            pltpu.VMEM((B, D), jnp.float32),
            pltpu.VMEM((6, B, HC), jnp.float32),
            pltpu.VMEM((B, D), jnp.float32),
            pltpu.VMEM((2, B, HC), jnp.float32),
            pltpu.VMEM((B, DFF), jnp.float32),
        ],
        compiler_params=pltpu.CompilerParams(
            dimension_semantics=("arbitrary",),
        ),
    )(h, win_r, wout_r, w1, w2, inb_r, outb_r, l1b_r, l2b_r, ln1w_r, ln1b_r,
      ln2w_r, ln2b_r, ow, ob2)


def kernel(inputs, edge_index, edge_attr, gcn_w1, gcn_b1, gcn_w2, gcn_b2,
           gcn_w3, gcn_b3, gcn_w4, gcn_b4, tr_in_w, tr_in_b, tr_out_w,
           tr_out_b, tr_l1_w, tr_l1_b, tr_l2_w, tr_l2_b, tr_ln1_w, tr_ln1_b,
           tr_ln2_w, tr_ln2_b, out_w, out_b):
    f32 = jnp.float32
    x3 = jnp.transpose(inputs, (2, 0, 1))
    x3 = jnp.pad(x3, ((0, NP - ENC), (0, 0), (0, FP - SEQ)))
    src_c = edge_index[0].astype(f32).reshape(E, 1)
    dst_c = edge_index[1].astype(f32).reshape(E, 1)
    ew_c = edge_attr.reshape(E, 1)

    def padw(w, b):
        o, ci = w.shape
        wp = jnp.pad(w, ((0, FP - o), (0, FP - ci)))
        bp = jnp.pad(b, (0, FP - o)).reshape(1, FP)
        return wp, bp

    w1p, b1p = padw(gcn_w1, gcn_b1)
    w2p, b2p = padw(gcn_w2, gcn_b2)
    w3p, b3p = padw(gcn_w3, gcn_b3)
    w4p, b4p = padw(gcn_w4, gcn_b4)

    z4 = _graph_gcn_call(x3, src_c, dst_c, ew_c, w1p, b1p, w2p, b2p, w3p,
                         b3p, w4p, b4p)
    h = z4[:ENC, :, :SEQ // 6].transpose(1, 0, 2).reshape(B, D)

    win_r = tr_in_w.reshape(NLAYERS * 6, HC, D)
    wout_r = tr_out_w.reshape(NLAYERS * 2, HC, D)
    out = _transformer_call(
        h, win_r, wout_r, tr_l1_w, tr_l2_w,
        tr_in_b.reshape(NLAYERS * 6, 1, HC), tr_out_b.reshape(NLAYERS * 2, 1, HC),
        tr_l1_b.reshape(NLAYERS, 1, DFF), tr_l2_b.reshape(NLAYERS, 1, D),
        tr_ln1_w.reshape(NLAYERS, 1, D), tr_ln1_b.reshape(NLAYERS, 1, D),
        tr_ln2_w.reshape(NLAYERS, 1, D), tr_ln2_b.reshape(NLAYERS, 1, D),
        out_w, out_b.reshape(1, LBL))
    return out.reshape(B, 1, LBL)
